# baseline (speedup 1.0000x reference)
import jax
import jax.numpy as jnp
from jax.experimental import pallas as pl
from jax.experimental.pallas import tpu as pltpu
from jax.experimental.pallas import tpu_sc as plsc

VOCAB = 1024
DIM = 64
N_ROWS = 32 * 576
BLOCK = 2304
NB = N_ROWS // BLOCK
GATHER_WINDOW = 256


def _tc_body(x_ref, et_ref, a2_ref, b2_ref,
             idx_ref, counts_ref, se_ref):
    x = x_ref[...]
    xe = jnp.dot(x, et_ref[...], preferred_element_type=jnp.float32)
    a2 = a2_ref[...]
    b2 = b2_ref[...]
    d = (a2 + b2) - 4.0 * xe

    mv = jnp.min(d, axis=1, keepdims=True)
    lane = jax.lax.broadcasted_iota(jnp.int32, d.shape, 1)
    idx = jnp.min(jnp.where(d == mv, lane, jnp.int32(2 ** 30)),
                  axis=1, keepdims=True)
    idx_ref[...] = idx

    onehot = lane == idx
    counts = jnp.sum(onehot.astype(jnp.float32), axis=0, keepdims=True)
    counts_ref[...] = counts[None]
    se = jnp.sum(a2) + jnp.sum(jnp.where(onehot, b2 - 2.0 * xe, 0.0))
    se_ref[...] = jnp.zeros((1, 1, 128), jnp.float32) + se


def _tc_quantize(x, et, a2, b2):
    return pl.pallas_call(
        _tc_body,
        grid=(NB,),
        in_specs=[
            pl.BlockSpec((BLOCK, DIM), lambda i: (i, 0)),
            pl.BlockSpec((DIM, VOCAB), lambda i: (0, 0)),
            pl.BlockSpec((BLOCK, 1), lambda i: (i, 0)),
            pl.BlockSpec((1, VOCAB), lambda i: (0, 0)),
        ],
        out_specs=[
            pl.BlockSpec((BLOCK, 1), lambda i: (i, 0)),
            pl.BlockSpec((1, 1, VOCAB), lambda i: (i, 0, 0)),
            pl.BlockSpec((1, 1, 128), lambda i: (i, 0, 0)),
        ],
        out_shape=[
            jax.ShapeDtypeStruct((N_ROWS, 1), jnp.int32),
            jax.ShapeDtypeStruct((NB, 1, VOCAB), jnp.float32),
            jax.ShapeDtypeStruct((NB, 1, 128), jnp.float32),
        ],
        compiler_params=pltpu.CompilerParams(
            dimension_semantics=("parallel",)),
    )(x, et, a2, b2)


def _tc_finalize_body(counts_ref, se_ref, loss_ref, perp_ref):
    counts = jnp.sum(counts_ref[...], axis=0)
    avg = counts / jnp.float32(N_ROWS)
    ent = jnp.sum(avg * jnp.log(avg + 1e-10), axis=1, keepdims=True)
    perp_ref[...] = jnp.exp(-ent)
    se = jnp.sum(se_ref[...][:, :, :1])
    loss_ref[...] = jnp.reshape(
        1.25 * se / jnp.float32(N_ROWS * DIM), (1, 1))


def _tc_finalize(counts_p, se_p):
    return pl.pallas_call(
        _tc_finalize_body,
        out_shape=[
            jax.ShapeDtypeStruct((1, 1), jnp.float32),
            jax.ShapeDtypeStruct((1, 1), jnp.float32),
        ],
    )(counts_p, se_p)


def _sc_gather(emb_padded, idx_flat):
    mesh = plsc.VectorSubcoreMesh(core_axis_name="core",
                                  subcore_axis_name="subcore")

    @pl.kernel(out_type=jax.ShapeDtypeStruct((N_ROWS, 128), jnp.float32),
               mesh=mesh)
    def k(emb_hbm, i_hbm, o_hbm):
        def body(i_vmem, o_vmem):
            pltpu.sync_copy(emb_hbm.at[i_vmem.at[0]], o_vmem)

        pltpu.emit_pipeline(
            body,
            grid=(N_ROWS // GATHER_WINDOW,),
            in_specs=[pl.BlockSpec((1, GATHER_WINDOW),
                                   index_map=lambda i: (0, i))],
            out_specs=[pl.BlockSpec((GATHER_WINDOW, 128),
                                    index_map=lambda i: (i, 0))],
            core_axis_name="subcore",
            dimension_semantics=(pltpu.PARALLEL,),
        )(i_hbm, o_hbm)

    return k(emb_padded, idx_flat)


def kernel(inputs, embedding):
    input_shape = inputs.shape
    x = inputs.reshape(-1, DIM)
    a2 = jnp.sum(x ** 2, axis=1, keepdims=True)
    b2 = jnp.sum(embedding ** 2, axis=1)

    idx2, counts_p, se_p = _tc_quantize(x, embedding.T, a2,
                                        b2.reshape(1, VOCAB))
    loss, perp = _tc_finalize(counts_p, se_p)
    idx = idx2.reshape(-1)
    emb_padded = jnp.pad(embedding, ((0, 0), (0, 128 - DIM)))
    quantized = _sc_gather(emb_padded, idx.reshape(1, N_ROWS))[:, :DIM]

    quantized_st = quantized.reshape(input_shape)
    enc_idx_out = idx.reshape(input_shape[0], input_shape[1])
    return (quantized_st, enc_idx_out, loss.reshape(()), perp.reshape(()))


# speedup vs baseline: 1.2684x; 1.2684x over previous
import jax
import jax.numpy as jnp
from jax.experimental import pallas as pl
from jax.experimental.pallas import tpu as pltpu
from jax.experimental.pallas import tpu_sc as plsc

VOCAB = 1024
DIM = 64
N_ROWS = 32 * 576
BLOCK = 2304
NB = N_ROWS // BLOCK
EBLOCK = 4608
NEB = N_ROWS // EBLOCK
GATHER_WINDOW = 256


def _tc_body(x_ref, e_ref, a2_ref, b2_ref, idx_ref, counts_ref):
    x = x_ref[...]
    xe = jax.lax.dot_general(x, e_ref[...], (((1,), (1,)), ((), ())),
                             preferred_element_type=jnp.float32)
    d = (a2_ref[...] + b2_ref[...]) - 4.0 * xe

    mv = jnp.min(d, axis=1, keepdims=True)
    lane = jax.lax.broadcasted_iota(jnp.int32, d.shape, 1)
    idx = jnp.min(jnp.where(d == mv, lane, jnp.int32(2 ** 30)),
                  axis=1, keepdims=True)
    idx_ref[...] = idx

    onehot = lane == idx
    counts = jnp.sum(onehot.astype(jnp.float32), axis=0, keepdims=True)
    counts_ref[...] = counts[None]


def _tc_quantize(x, emb, a2, b2):
    return pl.pallas_call(
        _tc_body,
        grid=(NB,),
        in_specs=[
            pl.BlockSpec((BLOCK, DIM), lambda i: (i, 0)),
            pl.BlockSpec((VOCAB, DIM), lambda i: (0, 0)),
            pl.BlockSpec((BLOCK, 1), lambda i: (i, 0)),
            pl.BlockSpec((1, VOCAB), lambda i: (0, 0)),
        ],
        out_specs=[
            pl.BlockSpec((BLOCK, 1), lambda i: (i, 0)),
            pl.BlockSpec((1, 1, VOCAB), lambda i: (i, 0, 0)),
        ],
        out_shape=[
            jax.ShapeDtypeStruct((N_ROWS, 1), jnp.int32),
            jax.ShapeDtypeStruct((NB, 1, VOCAB), jnp.float32),
        ],
    )(x, emb, a2, b2)


def _sc_gather(emb_padded, idx_flat):
    mesh = plsc.VectorSubcoreMesh(core_axis_name="core",
                                  subcore_axis_name="subcore")

    @pl.kernel(out_type=jax.ShapeDtypeStruct((N_ROWS, 128), jnp.float32),
               mesh=mesh)
    def k(emb_hbm, i_hbm, o_hbm):
        def body(i_vmem, o_vmem):
            pltpu.sync_copy(emb_hbm.at[i_vmem.at[0]], o_vmem)

        pltpu.emit_pipeline(
            body,
            grid=(N_ROWS // GATHER_WINDOW,),
            in_specs=[pl.BlockSpec((1, GATHER_WINDOW),
                                   index_map=lambda i: (0, i))],
            out_specs=[pl.BlockSpec((GATHER_WINDOW, 128),
                                    index_map=lambda i: (i, 0))],
            core_axis_name=("core", "subcore"),
            dimension_semantics=(pltpu.PARALLEL,),
        )(i_hbm, o_hbm)

    return k(emb_padded, idx_flat)


def _tc_epilogue_body(x_ref, qp_ref, counts_ref,
                      qst_ref, loss_ref, perp_ref, se_ref):
    step = pl.program_id(0)
    q = qp_ref[...][:, :DIM]
    qst_ref[...] = q
    diff = q - x_ref[...]
    se = jnp.sum(diff * diff)

    @pl.when(step == 0)
    def _():
        se_ref[0] = 0.0

    se_ref[0] += se

    @pl.when(step == NEB - 1)
    def _():
        counts = jnp.sum(counts_ref[...], axis=0)
        avg = counts / jnp.float32(N_ROWS)
        ent = jnp.sum(avg * jnp.log(avg + 1e-10), axis=1, keepdims=True)
        perp_ref[...] = jnp.exp(-ent)
        loss_ref[...] = jnp.reshape(
            1.25 * se_ref[0] / jnp.float32(N_ROWS * DIM), (1, 1))


def _tc_epilogue(x, q_padded, counts_p):
    return pl.pallas_call(
        _tc_epilogue_body,
        grid=(NEB,),
        in_specs=[
            pl.BlockSpec((EBLOCK, DIM), lambda i: (i, 0)),
            pl.BlockSpec((EBLOCK, 128), lambda i: (i, 0)),
            pl.BlockSpec((NB, 1, VOCAB), lambda i: (0, 0, 0)),
        ],
        out_specs=[
            pl.BlockSpec((EBLOCK, DIM), lambda i: (i, 0)),
            pl.BlockSpec((1, 1), lambda i: (0, 0)),
            pl.BlockSpec((1, 1), lambda i: (0, 0)),
        ],
        out_shape=[
            jax.ShapeDtypeStruct((N_ROWS, DIM), jnp.float32),
            jax.ShapeDtypeStruct((1, 1), jnp.float32),
            jax.ShapeDtypeStruct((1, 1), jnp.float32),
        ],
        scratch_shapes=[pltpu.SMEM((1,), jnp.float32)],
    )(x, q_padded, counts_p)


def kernel(inputs, embedding):
    input_shape = inputs.shape
    x = inputs.reshape(-1, DIM)
    a2 = jnp.sum(x ** 2, axis=1, keepdims=True)
    b2 = jnp.sum(embedding ** 2, axis=1)
    emb_padded = jnp.pad(embedding, ((0, 0), (0, 128 - DIM)))

    idx2, counts_p = _tc_quantize(x, embedding, a2, b2.reshape(1, VOCAB))
    idx = idx2.reshape(-1)
    q_padded = _sc_gather(emb_padded, idx.reshape(1, N_ROWS))
    quantized, loss, perp = _tc_epilogue(x, q_padded, counts_p)

    quantized_st = quantized.reshape(input_shape)
    enc_idx_out = idx.reshape(input_shape[0], input_shape[1])
    return (quantized_st, enc_idx_out, loss.reshape(()), perp.reshape(()))
